# baseline (speedup 1.0000x reference)
import functools

import jax
import jax.numpy as jnp
from jax import lax
from jax.experimental import pallas as pl
from jax.experimental.pallas import tpu as pltpu
from jax.experimental.pallas import tpu_sc as plsc

VOCAB = 1000000
EMBED = 64
B_ROWS = 4096
B_COLS = 50
CHUNK = 128

_info = plsc.get_sparse_core_info()
NC, NS = _info.num_cores, _info.num_subcores
NW = NC * NS
NBUF = 5
LANES = 16

_TBLK = 2048
_THALF = _TBLK // 2
_TGRID = -(-VOCAB // _TBLK)
_TROWS = _TGRID * _THALF


def _tc_pack_body(in_ref, out_ref):
    blk = in_ref[...]
    t = jnp.transpose(blk, (1, 0))
    out_ref[...] = jnp.concatenate([t[:_THALF], t[_THALF:]], axis=1)


_tc_pack = pl.pallas_call(
    _tc_pack_body,
    grid=(_TGRID,),
    in_specs=[pl.BlockSpec((EMBED, _TBLK), lambda i: (0, i))],
    out_specs=pl.BlockSpec((_THALF, 2 * EMBED), lambda i: (i, 0)),
    out_shape=jax.ShapeDtypeStruct((_TROWS, 2 * EMBED), jnp.float32),
)


def _make_kernel():
    mesh = plsc.VectorSubcoreMesh(core_axis_name="c", subcore_axis_name="s")

    @functools.partial(
        pl.kernel,
        mesh=mesh,
        compiler_params=pltpu.CompilerParams(use_tc_tiling_on_sc=False),
        out_type=(
            jax.ShapeDtypeStruct((B_COLS, B_ROWS, EMBED), jnp.float32),
            jax.ShapeDtypeStruct((B_COLS, B_ROWS, EMBED), jnp.float32),
        ),
        scratch_types=[
            pltpu.VMEM((B_COLS, CHUNK), jnp.int32),
            pltpu.VMEM((B_COLS, CHUNK), jnp.int32),
            pltpu.VMEM((NBUF, CHUNK, 2 * EMBED), jnp.float32),
            [pltpu.SemaphoreType.DMA] * NBUF,
        ],
    )
    def k(idx_hbm, table_hbm, outa_hbm, outb_hbm, idx_v, gidx_v, pair_v, sems):
        wid = lax.axis_index("s") * NC + lax.axis_index("c")
        b0 = pl.multiple_of(wid * CHUNK, CHUNK)

        pltpu.sync_copy(idx_hbm.at[:, pl.ds(b0, CHUNK)], idx_v)

        for j in range(B_COLS):
            for g in range(CHUNK // LANES):
                sl = pl.ds(g * LANES, LANES)
                v = idx_v[j, sl]
                gidx_v[j, sl] = (
                    lax.shift_left(lax.shift_right_logical(v, 11), 10)
                    + (v & (_THALF - 1)))

        for b in range(NBUF):
            pltpu.async_copy(table_hbm.at[gidx_v.at[b]], pair_v.at[b], sems[b])

        @pl.loop(0, B_COLS, step=NBUF)
        def _ring(s0):
            for b in range(NBUF):
                s = s0 + b
                pltpu.make_async_copy(table_hbm.at[gidx_v.at[s]], pair_v.at[b],
                                      sems[b]).wait()
                pltpu.sync_copy(pair_v.at[b, :, pl.ds(0, EMBED)],
                                outa_hbm.at[s].at[pl.ds(b0, CHUNK)])
                pltpu.sync_copy(pair_v.at[b, :, pl.ds(EMBED, EMBED)],
                                outb_hbm.at[s].at[pl.ds(b0, CHUNK)])
                nxt = s + NBUF

                @pl.when(nxt < B_COLS)
                def _():
                    pltpu.async_copy(table_hbm.at[gidx_v.at[nxt]], pair_v.at[b],
                                     sems[b])

    return k


_kernel_call = _make_kernel()


def kernel(inputs, embeddings):
    idx_t = jnp.transpose(inputs.astype(jnp.int32))
    table_pk = _tc_pack(jnp.transpose(embeddings))
    out_a, out_b = _kernel_call(idx_t, table_pk)
    half = (idx_t >> 10) & 1
    out_d = jnp.where(half[:, :, None] == 0, out_a, out_b)
    return jnp.transpose(out_d, (1, 0, 2))


# speedup vs baseline: 1.7137x; 1.7137x over previous
import functools

import jax
import jax.numpy as jnp
from jax import lax
from jax.experimental import pallas as pl
from jax.experimental.pallas import tpu as pltpu
from jax.experimental.pallas import tpu_sc as plsc

VOCAB = 1000000
EMBED = 64
B_ROWS = 4096
B_COLS = 50
CHUNK = 128

_info = plsc.get_sparse_core_info()
NC, NS = _info.num_cores, _info.num_subcores
NW = NC * NS
NBUF = 5
LANES = 16

_TBLK = 8192
_THALF = _TBLK // 2
_TGRID = -(-VOCAB // _TBLK)
_TROWS = _TGRID * _THALF


def _tc_pack_body(in_ref, out_ref):
    blk = in_ref[...]
    t = jnp.transpose(blk, (1, 0))
    out_ref[...] = jnp.concatenate([t[:_THALF], t[_THALF:]], axis=1)


_tc_pack = pl.pallas_call(
    _tc_pack_body,
    grid=(_TGRID,),
    in_specs=[pl.BlockSpec((EMBED, _TBLK), lambda i: (0, i))],
    out_specs=pl.BlockSpec((_THALF, 2 * EMBED), lambda i: (i, 0)),
    out_shape=jax.ShapeDtypeStruct((_TROWS, 2 * EMBED), jnp.float32),
)


def _make_kernel():
    mesh = plsc.VectorSubcoreMesh(core_axis_name="c", subcore_axis_name="s")

    @functools.partial(
        pl.kernel,
        mesh=mesh,
        compiler_params=pltpu.CompilerParams(use_tc_tiling_on_sc=False),
        out_type=jax.ShapeDtypeStruct((B_COLS, B_ROWS, 2 * EMBED), jnp.float32),
        scratch_types=[
            pltpu.VMEM((B_COLS, CHUNK), jnp.int32),
            pltpu.VMEM((B_COLS, CHUNK), jnp.int32),
            pltpu.VMEM((NBUF, CHUNK, 2 * EMBED), jnp.float32),
            [pltpu.SemaphoreType.DMA] * NBUF,
        ],
    )
    def k(idx_hbm, table_hbm, out_hbm, idx_v, gidx_v, pair_v, sems):
        wid = lax.axis_index("s") * NC + lax.axis_index("c")
        b0 = pl.multiple_of(wid * CHUNK, CHUNK)

        pltpu.sync_copy(idx_hbm.at[:, pl.ds(b0, CHUNK)], idx_v)

        for j in range(B_COLS):
            for g in range(CHUNK // LANES):
                sl = pl.ds(g * LANES, LANES)
                v = idx_v[j, sl]
                gidx_v[j, sl] = (
                    lax.shift_left(lax.shift_right_logical(v, 13), 12)
                    + (v & (_THALF - 1)))

        for b in range(NBUF):
            pltpu.async_copy(table_hbm.at[gidx_v.at[b]], pair_v.at[b], sems[b])

        @pl.loop(0, B_COLS, step=NBUF)
        def _ring(s0):
            for b in range(NBUF):
                s = s0 + b
                pltpu.make_async_copy(table_hbm.at[gidx_v.at[s]], pair_v.at[b],
                                      sems[b]).wait()
                pltpu.sync_copy(pair_v.at[b],
                                out_hbm.at[s].at[pl.ds(b0, CHUNK)])
                nxt = s + NBUF

                @pl.when(nxt < B_COLS)
                def _():
                    pltpu.async_copy(table_hbm.at[gidx_v.at[nxt]], pair_v.at[b],
                                     sems[b])

    return k


_kernel_call = _make_kernel()


def kernel(inputs, embeddings):
    idx_t = jnp.transpose(inputs.astype(jnp.int32))
    table_pk = _tc_pack(jnp.transpose(embeddings))
    out_p = _kernel_call(idx_t, table_pk)
    half = (idx_t >> 12) & 1
    out_d = jnp.where(half[:, :, None] == 0,
                      out_p[:, :, :EMBED], out_p[:, :, EMBED:])
    return jnp.transpose(out_d, (1, 0, 2))


# speedup vs baseline: 1.8360x; 1.0714x over previous
import functools

import jax
import jax.numpy as jnp
from jax import lax
from jax.experimental import pallas as pl
from jax.experimental.pallas import tpu as pltpu
from jax.experimental.pallas import tpu_sc as plsc

VOCAB = 1000000
EMBED = 64
B_ROWS = 4096
B_COLS = 50
CHUNK = 128

_info = plsc.get_sparse_core_info()
NC, NS = _info.num_cores, _info.num_subcores
NW = NC * NS
NBUF = 5
LANES = 16

_TBLK = 16384
_THALF = _TBLK // 2
_TGRID = -(-VOCAB // _TBLK)
_TROWS = _TGRID * _THALF


def _tc_pack_body(in_ref, out_ref):
    blk = in_ref[...]
    t = jnp.transpose(blk, (1, 0))
    out_ref[...] = jnp.concatenate([t[:_THALF], t[_THALF:]], axis=1)


_tc_pack = pl.pallas_call(
    _tc_pack_body,
    grid=(_TGRID,),
    in_specs=[pl.BlockSpec((EMBED, _TBLK), lambda i: (0, i))],
    out_specs=pl.BlockSpec((_THALF, 2 * EMBED), lambda i: (i, 0)),
    out_shape=jax.ShapeDtypeStruct((_TROWS, 2 * EMBED), jnp.float32),
)


def _make_kernel():
    mesh = plsc.VectorSubcoreMesh(core_axis_name="c", subcore_axis_name="s")

    @functools.partial(
        pl.kernel,
        mesh=mesh,
        compiler_params=pltpu.CompilerParams(use_tc_tiling_on_sc=False),
        out_type=jax.ShapeDtypeStruct((B_COLS, B_ROWS, 2 * EMBED), jnp.float32),
        scratch_types=[
            pltpu.VMEM((B_COLS, CHUNK), jnp.int32),
            pltpu.VMEM((B_COLS, CHUNK), jnp.int32),
            pltpu.VMEM((NBUF, CHUNK, 2 * EMBED), jnp.float32),
            [pltpu.SemaphoreType.DMA] * NBUF,
        ],
    )
    def k(idx_hbm, table_hbm, out_hbm, idx_v, gidx_v, pair_v, sems):
        wid = lax.axis_index("s") * NC + lax.axis_index("c")
        b0 = pl.multiple_of(wid * CHUNK, CHUNK)

        pltpu.sync_copy(idx_hbm.at[:, pl.ds(b0, CHUNK)], idx_v)

        for j in range(B_COLS):
            for g in range(CHUNK // LANES):
                sl = pl.ds(g * LANES, LANES)
                v = idx_v[j, sl]
                gidx_v[j, sl] = (
                    lax.shift_left(lax.shift_right_logical(v, 14), 13)
                    + (v & (_THALF - 1)))

        for b in range(NBUF):
            pltpu.async_copy(table_hbm.at[gidx_v.at[b]], pair_v.at[b], sems[b])

        @pl.loop(0, B_COLS, step=NBUF)
        def _ring(s0):
            for b in range(NBUF):
                s = s0 + b
                pltpu.make_async_copy(table_hbm.at[gidx_v.at[s]], pair_v.at[b],
                                      sems[b]).wait()
                pltpu.sync_copy(pair_v.at[b],
                                out_hbm.at[s].at[pl.ds(b0, CHUNK)])
                nxt = s + NBUF

                @pl.when(nxt < B_COLS)
                def _():
                    pltpu.async_copy(table_hbm.at[gidx_v.at[nxt]], pair_v.at[b],
                                     sems[b])

    return k


_kernel_call = _make_kernel()


def kernel(inputs, embeddings):
    idx_t = jnp.transpose(inputs.astype(jnp.int32))
    table_pk = _tc_pack(jnp.transpose(embeddings))
    out_p = _kernel_call(idx_t, table_pk)
    half = (idx_t >> 13) & 1
    out_d = jnp.where(half[:, :, None] == 0,
                      out_p[:, :, :EMBED], out_p[:, :, EMBED:])
    return jnp.transpose(out_d, (1, 0, 2))
